# Initial kernel scaffold; baseline (speedup 1.0000x reference)
import jax
import jax.numpy as jnp
from jax.experimental import pallas as pl


def kernel(support_features, support_labels, query_features):
    raise NotImplementedError("write your pallas kernel here")


# speedup vs baseline: 9.2833x; 9.2833x over previous
import functools

import jax
import jax.numpy as jnp
from jax import lax
from jax.experimental import pallas as pl
from jax.experimental.pallas import tpu as pltpu

NUM_CLASSES = 64
FDIM = 128


def _segsum_body(labels_ref, feat_ref, sums_ref, counts_ref):
    i = pl.program_id(0)

    labels = labels_ref[0, 0, :]
    feats = feat_ref[...]
    onehot = (
        lax.broadcasted_iota(jnp.int32, (NUM_CLASSES, labels.shape[0]), 0)
        == labels[None, :]
    ).astype(jnp.float32)
    partial = jnp.dot(onehot, feats, preferred_element_type=jnp.float32)
    cnt = jnp.sum(onehot, axis=1)

    @pl.when(i == 0)
    def _():
        sums_ref[...] = jnp.zeros_like(sums_ref)
        counts_ref[...] = jnp.zeros_like(counts_ref)

    sums_ref[...] += partial
    counts_ref[...] += jnp.broadcast_to(cnt[:, None], counts_ref.shape)


def _segment_sums(support_features, support_labels, block_rows):
    n = support_features.shape[0]
    assert n % block_rows == 0
    nblocks = n // block_rows
    labels3d = support_labels.reshape(nblocks, 1, block_rows)
    return pl.pallas_call(
        _segsum_body,
        grid=(nblocks,),
        in_specs=[
            pl.BlockSpec((1, 1, block_rows), lambda i: (i, 0, 0)),
            pl.BlockSpec((block_rows, FDIM), lambda i: (i, 0)),
        ],
        out_specs=[
            pl.BlockSpec((NUM_CLASSES, FDIM), lambda i: (0, 0)),
            pl.BlockSpec((NUM_CLASSES, FDIM), lambda i: (0, 0)),
        ],
        out_shape=[
            jax.ShapeDtypeStruct((NUM_CLASSES, FDIM), jnp.float32),
            jax.ShapeDtypeStruct((NUM_CLASSES, FDIM), jnp.float32),
        ],
    )(labels3d, support_features)


def _distance_body(sums_ref, counts_ref, q_ref, out_ref):
    cnt = counts_ref[:, 0]
    present = cnt > 0.0
    tri = (
        lax.broadcasted_iota(jnp.int32, (NUM_CLASSES, NUM_CLASSES), 0)
        < lax.broadcasted_iota(jnp.int32, (NUM_CLASSES, NUM_CLASSES), 1)
    )
    rank = jnp.sum((present[:, None] & tri).astype(jnp.int32), axis=0)
    perm = (
        (lax.broadcasted_iota(jnp.int32, (NUM_CLASSES, NUM_CLASSES), 0)
         == rank[None, :])
        & present[None, :]
    ).astype(jnp.float32)
    protos_by_label = sums_ref[...] / jnp.maximum(counts_ref[...], 1.0)
    protos = jnp.dot(perm, protos_by_label, preferred_element_type=jnp.float32)

    q = q_ref[...]
    qsq = jnp.sum(q * q, axis=1, keepdims=True)
    psq = jnp.sum(protos * protos, axis=1)
    cross = lax.dot_general(
        q, protos, (((1,), (1,)), ((), ())),
        preferred_element_type=jnp.float32,
    )
    dist = jnp.maximum(qsq + psq[None, :] - 2.0 * cross, 0.0)
    logits = -dist
    m = jnp.max(logits, axis=1, keepdims=True)
    shifted = logits - m
    lse = jnp.log(jnp.sum(jnp.exp(shifted), axis=1, keepdims=True))
    out_ref[...] = shifted - lse


def _distances(sums, counts, query_features, block_rows):
    nq = query_features.shape[0]
    assert nq % block_rows == 0
    nblocks = nq // block_rows
    return pl.pallas_call(
        _distance_body,
        grid=(nblocks,),
        in_specs=[
            pl.BlockSpec((NUM_CLASSES, FDIM), lambda i: (0, 0)),
            pl.BlockSpec((NUM_CLASSES, FDIM), lambda i: (0, 0)),
            pl.BlockSpec((block_rows, FDIM), lambda i: (i, 0)),
        ],
        out_specs=pl.BlockSpec((block_rows, NUM_CLASSES), lambda i: (i, 0)),
        out_shape=jax.ShapeDtypeStruct((nq, NUM_CLASSES), jnp.float32),
    )(sums, counts, query_features)


@functools.partial(jax.jit, static_argnames=())
def kernel(support_features, support_labels, query_features):
    sums, counts = _segment_sums(support_features, support_labels, 2560)
    return _distances(sums, counts, query_features, 2000)
